# Initial kernel scaffold; baseline (speedup 1.0000x reference)
import jax
import jax.numpy as jnp
from jax.experimental import pallas as pl


def kernel(x, edge_index, W1, b1, W2, b2):
    raise NotImplementedError("write your pallas kernel here")


# speedup vs baseline: 36.9153x; 36.9153x over previous
import functools

import jax
import jax.numpy as jnp
from jax import lax
from jax.experimental import pallas as pl
from jax.experimental.pallas import tpu as pltpu
from jax.experimental.pallas import tpu_sc as plsc

NC = 2
NS = 16
NW = NC * NS
LANES = 16
CH = 128
NBUF = 4
D_HID = 16
ROWBLK = 1024


def _mesh():
    return plsc.VectorSubcoreMesh(
        core_axis_name="c", subcore_axis_name="s", num_cores=NC, num_subcores=NS
    )


def _make_deg_kernel(nch, n_pad, rpt):
    @functools.partial(
        pl.kernel,
        out_type=jax.ShapeDtypeStruct((NC, n_pad), jnp.float32),
        mesh=_mesh(),
        scratch_types=[
            pltpu.VMEM((nch, CH), jnp.int32),
            pltpu.VMEM((CH,), jnp.float32),
            pltpu.VMEM((CH,), jnp.float32),
            pltpu.VMEM_SHARED((n_pad,), jnp.float32),
        ],
    )
    def deg_kernel(dst_hbm, out_hbm, idx_v, ones_v, zero_v, acc):
        cid = lax.axis_index("c")
        sid = lax.axis_index("s")
        wid = sid * NC + cid
        for i in range(CH // LANES):
            ones_v[pl.ds(LANES * i, LANES)] = jnp.full((LANES,), 1.0, jnp.float32)
            zero_v[pl.ds(LANES * i, LANES)] = jnp.zeros((LANES,), jnp.float32)
        pltpu.sync_copy(dst_hbm.at[wid], idx_v)
        base = sid * rpt
        for t in range(rpt // CH):
            pltpu.sync_copy(zero_v, acc.at[pl.ds(base + t * CH, CH)])
        plsc.subcore_barrier()

        def chunk(j, carry):
            pltpu.sync_copy(ones_v, acc.at[idx_v.at[j]], add=True)
            return carry

        lax.fori_loop(0, nch, chunk, 0)
        plsc.subcore_barrier()
        pltpu.sync_copy(acc.at[pl.ds(base, rpt)], out_hbm.at[cid, pl.ds(base, rpt)])

    return deg_kernel


def _make_agg_kernel(nch, n_pad, rpt):
    @functools.partial(
        pl.kernel,
        out_type=jax.ShapeDtypeStruct((NC, n_pad, D_HID), jnp.float32),
        mesh=_mesh(),
        scratch_types=[
            pltpu.VMEM((nch, CH), jnp.int32),
            pltpu.VMEM((nch, CH), jnp.int32),
            pltpu.VMEM((NBUF, CH, D_HID), jnp.float32),
            pltpu.VMEM_SHARED((n_pad, D_HID), jnp.float32),
        ]
        + [pltpu.SemaphoreType.DMA] * NBUF,
        compiler_params=pltpu.CompilerParams(use_tc_tiling_on_sc=False),
        name="gcn_row_agg",
    )
    def agg_kernel(table_hbm, src_hbm, dst_hbm, out_hbm, src_v, dst_v, rows_v,
                   acc, *sems):
        cid = lax.axis_index("c")
        sid = lax.axis_index("s")
        wid = sid * NC + cid
        base = sid * rpt

        pltpu.sync_copy(src_hbm.at[wid], src_v)
        pltpu.sync_copy(dst_hbm.at[wid], dst_v)

        def zrow(i, carry):
            rows_v[0, i, :] = jnp.zeros((LANES,), jnp.float32)
            return carry

        lax.fori_loop(0, CH, zrow, 0)
        for t in range(rpt // CH):
            pltpu.sync_copy(rows_v.at[0], acc.at[pl.ds(base + t * CH, CH)])
        plsc.subcore_barrier()

        for b in range(NBUF):
            pltpu.async_copy(table_hbm.at[src_v.at[b]], rows_v.at[b], sems[b])

        def group(g, carry):
            for b in range(NBUF):
                j = g * NBUF + b
                pltpu.make_async_copy(
                    table_hbm.at[src_v.at[j]], rows_v.at[b], sems[b]
                ).wait()
                pltpu.sync_copy(rows_v.at[b], acc.at[dst_v.at[j]], add=True)

                @pl.when(j + NBUF < nch)
                def _():
                    pltpu.async_copy(
                        table_hbm.at[src_v.at[j + NBUF]], rows_v.at[b], sems[b]
                    )

            return carry

        lax.fori_loop(0, nch // NBUF, group, 0)
        plsc.subcore_barrier()
        pltpu.sync_copy(
            acc.at[pl.ds(base, rpt)], out_hbm.at[cid, pl.ds(base, rpt)]
        )

    return agg_kernel


def _tc_a_body(x_ref, w_ref, degp_ref, zp_ref, dinv_ref):
    z = jnp.dot(x_ref[...], w_ref[...], preferred_element_type=jnp.float32)
    deg = 1.0 + degp_ref[0] + degp_ref[1]
    dinv = 1.0 / jnp.sqrt(deg)
    dinv_ref[...] = dinv
    zp_ref[...] = z * dinv


def _tc_a(x_pad, W1, degp):
    n_pad, d_in = x_pad.shape
    grid = n_pad // ROWBLK
    return pl.pallas_call(
        _tc_a_body,
        grid=(grid,),
        in_specs=[
            pl.BlockSpec((ROWBLK, d_in), lambda i: (i, 0)),
            pl.BlockSpec((d_in, D_HID), lambda i: (0, 0)),
            pl.BlockSpec((NC, ROWBLK, 1), lambda i: (0, i, 0)),
        ],
        out_specs=[
            pl.BlockSpec((ROWBLK, D_HID), lambda i: (i, 0)),
            pl.BlockSpec((ROWBLK, 1), lambda i: (i, 0)),
        ],
        out_shape=[
            jax.ShapeDtypeStruct((n_pad, D_HID), jnp.float32),
            jax.ShapeDtypeStruct((n_pad, 1), jnp.float32),
        ],
    )(x_pad, W1, degp)


def _tc_b_body(agg_ref, zp_ref, dinv_ref, b1_ref, out_ref):
    s = agg_ref[0] + agg_ref[1] + zp_ref[...]
    dinv = dinv_ref[...]
    h1 = jnp.maximum(dinv * s + b1_ref[...], 0.0)
    out_ref[...] = h1 * dinv


def _tc_b(agg1, zp, dinv, b1):
    n_pad = zp.shape[0]
    grid = n_pad // ROWBLK
    return pl.pallas_call(
        _tc_b_body,
        grid=(grid,),
        in_specs=[
            pl.BlockSpec((NC, ROWBLK, D_HID), lambda i: (0, i, 0)),
            pl.BlockSpec((ROWBLK, D_HID), lambda i: (i, 0)),
            pl.BlockSpec((ROWBLK, 1), lambda i: (i, 0)),
            pl.BlockSpec((1, D_HID), lambda i: (0, 0)),
        ],
        out_specs=pl.BlockSpec((ROWBLK, D_HID), lambda i: (i, 0)),
        out_shape=jax.ShapeDtypeStruct((n_pad, D_HID), jnp.float32),
    )(agg1, zp, dinv, b1)


def _tc_c_body(agg_ref, h1p_ref, dinv_ref, w2_ref, b2_ref, out_ref):
    s = agg_ref[0] + agg_ref[1] + h1p_ref[...]
    pre = dinv_ref[...] * s
    h2 = jnp.dot(pre, w2_ref[...], preferred_element_type=jnp.float32)
    h2 = h2 + b2_ref[...]
    m = jnp.max(h2, axis=1, keepdims=True)
    e = jnp.exp(h2 - m)
    lse = jnp.log(jnp.sum(e, axis=1, keepdims=True))
    out_ref[...] = h2 - m - lse


def _tc_c(agg2, h1p, dinv, W2, b2):
    n_pad = h1p.shape[0]
    n_cls = W2.shape[1]
    grid = n_pad // ROWBLK
    return pl.pallas_call(
        _tc_c_body,
        grid=(grid,),
        in_specs=[
            pl.BlockSpec((NC, ROWBLK, D_HID), lambda i: (0, i, 0)),
            pl.BlockSpec((ROWBLK, D_HID), lambda i: (i, 0)),
            pl.BlockSpec((ROWBLK, 1), lambda i: (i, 0)),
            pl.BlockSpec((D_HID, n_cls), lambda i: (0, 0)),
            pl.BlockSpec((1, n_cls), lambda i: (0, 0)),
        ],
        out_specs=pl.BlockSpec((ROWBLK, n_cls), lambda i: (i, 0)),
        out_shape=jax.ShapeDtypeStruct((n_pad, n_cls), jnp.float32),
    )(agg2, h1p, dinv, W2, b2)


def kernel(x, edge_index, W1, b1, W2, b2):
    n = x.shape[0]
    e = edge_index.shape[1]

    epw = -(-e // (NW * CH * NBUF)) * CH * NBUF
    nch = epw // CH
    e_pad = NW * epw
    n_pad = -(-(n + 1) // (NS * CH)) * (NS * CH)
    rpt = n_pad // NS

    src = edge_index[0].astype(jnp.int32)
    dst = edge_index[1].astype(jnp.int32)
    pad = e_pad - e
    src3 = jnp.concatenate([src, jnp.zeros((pad,), jnp.int32)]).reshape(NW, nch, CH)
    dst3 = jnp.concatenate([dst, jnp.full((pad,), n, jnp.int32)]).reshape(NW, nch, CH)

    degp = _make_deg_kernel(nch, n_pad, rpt)(dst3)
    x_pad = jnp.pad(x, ((0, n_pad - n), (0, 0)))

    zp, dinv = _tc_a(x_pad, W1, degp.reshape(NC, n_pad, 1))
    agg = _make_agg_kernel(nch, n_pad, rpt)
    agg1 = agg(zp, src3, dst3)
    h1p = _tc_b(agg1, zp, dinv, b1.reshape(1, D_HID))
    agg2 = agg(h1p, src3, dst3)
    out = _tc_c(agg2, h1p, dinv, W2, b2.reshape(1, -1))
    return out[:n]


# speedup vs baseline: 38.4974x; 1.0429x over previous
import functools

import jax
import jax.numpy as jnp
from jax import lax
from jax.experimental import pallas as pl
from jax.experimental.pallas import tpu as pltpu
from jax.experimental.pallas import tpu_sc as plsc

NC = 2
NS = 16
NW = NC * NS
LANES = 16
CH = 128
NBUF = 8
D_HID = 16
ROWBLK = 1000


def _mesh():
    return plsc.VectorSubcoreMesh(
        core_axis_name="c", subcore_axis_name="s", num_cores=NC, num_subcores=NS
    )


def _make_deg_kernel(nch, n_pad, rpt):
    @functools.partial(
        pl.kernel,
        out_type=jax.ShapeDtypeStruct((NC, n_pad), jnp.float32),
        mesh=_mesh(),
        scratch_types=[
            pltpu.VMEM((nch, CH), jnp.int32),
            pltpu.VMEM((CH,), jnp.float32),
            pltpu.VMEM((CH,), jnp.float32),
            pltpu.VMEM_SHARED((n_pad,), jnp.float32),
        ]
        + [pltpu.SemaphoreType.DMA] * NBUF,
    )
    def deg_kernel(dst_hbm, out_hbm, idx_v, ones_v, zero_v, acc, *sems):
        cid = lax.axis_index("c")
        sid = lax.axis_index("s")
        wid = sid * NC + cid
        for i in range(CH // LANES):
            ones_v[pl.ds(LANES * i, LANES)] = jnp.full((LANES,), 1.0, jnp.float32)
            zero_v[pl.ds(LANES * i, LANES)] = jnp.zeros((LANES,), jnp.float32)
        pltpu.sync_copy(dst_hbm.at[wid], idx_v)
        base = sid * rpt
        for t in range(rpt // CH):
            pltpu.sync_copy(zero_v, acc.at[pl.ds(base + t * CH, CH)])
        plsc.subcore_barrier()

        for b in range(NBUF):
            pltpu.async_copy(ones_v, acc.at[idx_v.at[b]], sems[b], add=True)

        def group(g, carry):
            for b in range(NBUF):
                j = g * NBUF + b
                pltpu.make_async_copy(ones_v, acc.at[idx_v.at[j]], sems[b]).wait()

                @pl.when(j + NBUF < nch)
                def _():
                    pltpu.async_copy(
                        ones_v, acc.at[idx_v.at[j + NBUF]], sems[b], add=True
                    )

            return carry

        lax.fori_loop(0, nch // NBUF, group, 0)
        plsc.subcore_barrier()
        pltpu.sync_copy(acc.at[pl.ds(base, rpt)], out_hbm.at[cid, pl.ds(base, rpt)])

    return deg_kernel


def _make_agg_kernel(nch, n_pad, rpt):
    @functools.partial(
        pl.kernel,
        out_type=jax.ShapeDtypeStruct((NC, n_pad, D_HID), jnp.float32),
        mesh=_mesh(),
        scratch_types=[
            pltpu.VMEM((nch, CH), jnp.int32),
            pltpu.VMEM((nch, CH), jnp.int32),
            pltpu.VMEM((NBUF, CH, D_HID), jnp.float32),
            pltpu.VMEM_SHARED((n_pad, D_HID), jnp.float32),
        ]
        + [pltpu.SemaphoreType.DMA] * (2 * NBUF),
        compiler_params=pltpu.CompilerParams(use_tc_tiling_on_sc=False),
        name="gcn_row_agg",
    )
    def agg_kernel(table_hbm, src_hbm, dst_hbm, out_hbm, src_v, dst_v, rows_v,
                   acc, *sems):
        cid = lax.axis_index("c")
        sid = lax.axis_index("s")
        wid = sid * NC + cid
        base = sid * rpt

        pltpu.sync_copy(src_hbm.at[wid], src_v)
        pltpu.sync_copy(dst_hbm.at[wid], dst_v)

        def zrow(i, carry):
            rows_v[0, i, :] = jnp.zeros((LANES,), jnp.float32)
            return carry

        lax.fori_loop(0, CH, zrow, 0)
        for t in range(rpt // CH):
            pltpu.sync_copy(rows_v.at[0], acc.at[pl.ds(base + t * CH, CH)])
        plsc.subcore_barrier()

        gsems = sems[:NBUF]
        ssems = sems[NBUF:]

        for b in range(NBUF):
            pltpu.async_copy(table_hbm.at[src_v.at[b]], rows_v.at[b], gsems[b])

        def group(g, carry):
            for b in range(NBUF):
                j = g * NBUF + b
                pltpu.make_async_copy(
                    table_hbm.at[src_v.at[j]], rows_v.at[b], gsems[b]
                ).wait()
                pltpu.async_copy(
                    rows_v.at[b], acc.at[dst_v.at[j]], ssems[b], add=True
                )
            for b in range(NBUF):
                j = g * NBUF + b
                pltpu.make_async_copy(
                    rows_v.at[b], acc.at[dst_v.at[j]], ssems[b]
                ).wait()

                @pl.when(j + NBUF < nch)
                def _():
                    pltpu.async_copy(
                        table_hbm.at[src_v.at[j + NBUF]], rows_v.at[b], gsems[b]
                    )

            return carry

        lax.fori_loop(0, nch // NBUF, group, 0)
        plsc.subcore_barrier()
        pltpu.sync_copy(
            acc.at[pl.ds(base, rpt)], out_hbm.at[cid, pl.ds(base, rpt)]
        )

    return agg_kernel


def _tc_a_body(x_ref, w_ref, degp_ref, zp_ref, dinv_ref):
    z = jnp.dot(x_ref[...], w_ref[...], preferred_element_type=jnp.float32)
    deg = 1.0 + degp_ref[0] + degp_ref[1]
    dinv = 1.0 / jnp.sqrt(deg)
    dinv_ref[...] = dinv
    zp_ref[...] = z * dinv


def _tc_a(x, W1, degp):
    n, d_in = x.shape
    grid = n // ROWBLK
    return pl.pallas_call(
        _tc_a_body,
        grid=(grid,),
        in_specs=[
            pl.BlockSpec((ROWBLK, d_in), lambda i: (i, 0)),
            pl.BlockSpec((d_in, D_HID), lambda i: (0, 0)),
            pl.BlockSpec((NC, ROWBLK, 1), lambda i: (0, i, 0)),
        ],
        out_specs=[
            pl.BlockSpec((ROWBLK, D_HID), lambda i: (i, 0)),
            pl.BlockSpec((ROWBLK, 1), lambda i: (i, 0)),
        ],
        out_shape=[
            jax.ShapeDtypeStruct((n, D_HID), jnp.float32),
            jax.ShapeDtypeStruct((n, 1), jnp.float32),
        ],
    )(x, W1, degp)


def _tc_b_body(agg_ref, zp_ref, dinv_ref, b1_ref, out_ref):
    s = agg_ref[0] + agg_ref[1] + zp_ref[...]
    dinv = dinv_ref[...]
    h1 = jnp.maximum(dinv * s + b1_ref[...], 0.0)
    out_ref[...] = h1 * dinv


def _tc_b(agg1, zp, dinv, b1):
    n = zp.shape[0]
    grid = n // ROWBLK
    return pl.pallas_call(
        _tc_b_body,
        grid=(grid,),
        in_specs=[
            pl.BlockSpec((NC, ROWBLK, D_HID), lambda i: (0, i, 0)),
            pl.BlockSpec((ROWBLK, D_HID), lambda i: (i, 0)),
            pl.BlockSpec((ROWBLK, 1), lambda i: (i, 0)),
            pl.BlockSpec((1, D_HID), lambda i: (0, 0)),
        ],
        out_specs=pl.BlockSpec((ROWBLK, D_HID), lambda i: (i, 0)),
        out_shape=jax.ShapeDtypeStruct((n, D_HID), jnp.float32),
    )(agg1, zp, dinv, b1)


def _tc_c_body(agg_ref, h1p_ref, dinv_ref, w2_ref, b2_ref, out_ref):
    s = agg_ref[0] + agg_ref[1] + h1p_ref[...]
    pre = dinv_ref[...] * s
    h2 = jnp.dot(pre, w2_ref[...], preferred_element_type=jnp.float32)
    h2 = h2 + b2_ref[...]
    m = jnp.max(h2, axis=1, keepdims=True)
    e = jnp.exp(h2 - m)
    lse = jnp.log(jnp.sum(e, axis=1, keepdims=True))
    out_ref[...] = h2 - m - lse


def _tc_c(agg2, h1p, dinv, W2, b2):
    n = h1p.shape[0]
    n_cls = W2.shape[1]
    grid = n // ROWBLK
    return pl.pallas_call(
        _tc_c_body,
        grid=(grid,),
        in_specs=[
            pl.BlockSpec((NC, ROWBLK, D_HID), lambda i: (0, i, 0)),
            pl.BlockSpec((ROWBLK, D_HID), lambda i: (i, 0)),
            pl.BlockSpec((ROWBLK, 1), lambda i: (i, 0)),
            pl.BlockSpec((D_HID, n_cls), lambda i: (0, 0)),
            pl.BlockSpec((1, n_cls), lambda i: (0, 0)),
        ],
        out_specs=pl.BlockSpec((ROWBLK, n_cls), lambda i: (i, 0)),
        out_shape=jax.ShapeDtypeStruct((n, n_cls), jnp.float32),
    )(agg2, h1p, dinv, W2, b2)


def kernel(x, edge_index, W1, b1, W2, b2):
    n = x.shape[0]
    e = edge_index.shape[1]

    epw = -(-e // (NW * CH * NBUF)) * CH * NBUF
    nch = epw // CH
    e_pad = NW * epw
    n_pad = -(-(n + 1) // (NS * CH)) * (NS * CH)
    rpt = n_pad // NS

    src = edge_index[0].astype(jnp.int32)
    dst = edge_index[1].astype(jnp.int32)
    pad = e_pad - e
    src3 = jnp.concatenate([src, jnp.zeros((pad,), jnp.int32)]).reshape(NW, nch, CH)
    dst3 = jnp.concatenate([dst, jnp.full((pad,), n, jnp.int32)]).reshape(NW, nch, CH)

    degp = _make_deg_kernel(nch, n_pad, rpt)(dst3)

    zp, dinv = _tc_a(x, W1, degp.reshape(NC, n_pad, 1))
    agg = _make_agg_kernel(nch, n_pad, rpt)
    agg1 = agg(zp, src3, dst3)
    h1p = _tc_b(agg1, zp, dinv, b1.reshape(1, D_HID))
    agg2 = agg(h1p, src3, dst3)
    return _tc_c(agg2, h1p, dinv, W2, b2.reshape(1, -1))
